# baseline (speedup 1.0000x reference)
import functools

import jax
import jax.numpy as jnp
from jax import lax
from jax.experimental import pallas as pl
from jax.experimental.pallas import tpu as pltpu
from jax.experimental.pallas import tpu_sc as plsc

_info = plsc.get_sparse_core_info()
_NC, _NS, _L = _info.num_cores, _info.num_subcores, _info.num_lanes
_NW = _NC * _NS


def _make_transform(F, V, D):
    TILE = 128
    JFULL = V // TILE
    VTAIL = V - JFULL * TILE
    NB = F * JFULL
    per_w = -(-NB // _NW)
    per_w += per_w % 2
    n_steps = per_w // 2

    mesh = plsc.VectorSubcoreMesh(core_axis_name="c", subcore_axis_name="s")

    @functools.partial(
        pl.kernel,
        mesh=mesh,
        out_type=jax.ShapeDtypeStruct((F * V * D,), jnp.float32),
        compiler_params=pltpu.CompilerParams(needs_layout_passes=False),
        scratch_types=[
            pltpu.VMEM((D, TILE), jnp.float32),
            pltpu.VMEM((D, TILE), jnp.float32),
            pltpu.VMEM((TILE * D,), jnp.float32),
            pltpu.VMEM((TILE * D,), jnp.float32),
            pltpu.VMEM((D, VTAIL), jnp.float32),
            pltpu.VMEM((VTAIL * D,), jnp.float32),
            pltpu.SemaphoreType.DMA,
            pltpu.SemaphoreType.DMA,
            pltpu.SemaphoreType.DMA,
            pltpu.SemaphoreType.DMA,
        ],
    )
    def transform_kernel(tab_t, out_flat, s0, s1, o0, o1, st, ot,
                         isem0, isem1, osem0, osem1):
        wid = lax.axis_index("s") * _NC + lax.axis_index("c")
        g0 = wid * per_w
        lanes = lax.iota(jnp.int32, _L)
        zeros = lanes - lanes
        i_lo = lanes
        i_hi = lanes + _L

        def src_of(g):
            g = jnp.minimum(g, NB - 1)
            f = g // JFULL
            j = g - f * JFULL
            return tab_t.at[f, :, pl.ds(j * TILE, TILE)]

        def dst_of(g):
            g = jnp.minimum(g, NB - 1)
            f = g // JFULL
            j = g - f * JFULL
            return out_flat.at[pl.ds(f * V * D + j * TILE * D, TILE * D)]

        def transpose_block(s, o, ncols):
            for rr in range(ncols):
                cvec = zeros + rr
                a = plsc.load_gather(s, [i_lo, cvec])
                b = plsc.load_gather(s, [i_hi, cvec])
                o[pl.ds(rr * D, _L)] = a
                o[pl.ds(rr * D + _L, _L)] = b

        pltpu.async_copy(src_of(g0), s0, isem0)

        def step(u, carry):
            g_e = g0 + 2 * u
            g_o = g_e + 1
            g_n = g_e + 2

            pltpu.async_copy(src_of(g_o), s1, isem1)
            pltpu.make_async_copy(src_of(g_e), s0, isem0).wait()

            @pl.when(u > 0)
            def _():
                pltpu.make_async_copy(o0, dst_of(g_e), osem0).wait()

            transpose_block(s0, o0, TILE)
            pltpu.async_copy(o0, dst_of(g_e), osem0)

            pltpu.async_copy(src_of(g_n), s0, isem0)
            pltpu.make_async_copy(src_of(g_o), s1, isem1).wait()

            @pl.when(u > 0)
            def _():
                pltpu.make_async_copy(o1, dst_of(g_o), osem1).wait()

            transpose_block(s1, o1, TILE)
            pltpu.async_copy(o1, dst_of(g_o), osem1)

            return carry

        lax.fori_loop(0, n_steps, step, 0)

        pltpu.make_async_copy(src_of(g0), s0, isem0).wait()
        pltpu.make_async_copy(o0, dst_of(g0), osem0).wait()
        pltpu.make_async_copy(o1, dst_of(g0), osem1).wait()

        wt = jnp.minimum(wid, F - 1)
        pltpu.sync_copy(tab_t.at[wt, :, pl.ds(JFULL * TILE, VTAIL)], st)
        transpose_block(st, ot, VTAIL)
        pltpu.sync_copy(
            ot,
            out_flat.at[pl.ds(wt * V * D + JFULL * TILE * D, VTAIL * D)],
        )

    return transform_kernel


def _make_gather(F, V, D, N):
    assert N % _NW == 0
    per_w = N // _NW
    CHUNK = 1664
    ROWS = CHUNK // 128
    PERIOD = 13 * _L
    assert per_w % CHUNK == 0 and per_w % PERIOD == 0
    n_chunks = per_w // CHUNK

    mesh = plsc.VectorSubcoreMesh(core_axis_name="c", subcore_axis_name="s")

    @functools.partial(
        pl.kernel,
        mesh=mesh,
        out_type=jax.ShapeDtypeStruct((N, D), jnp.float32),
        compiler_params=pltpu.CompilerParams(use_tc_tiling_on_sc=False),
        scratch_types=[
            pltpu.VMEM((per_w,), jnp.int32),
            pltpu.VMEM((PERIOD,), jnp.int32),
            pltpu.VMEM((CHUNK, D), jnp.float32),
            pltpu.VMEM((CHUNK, D), jnp.float32),
            pltpu.SemaphoreType.DMA,
            pltpu.SemaphoreType.DMA,
            pltpu.SemaphoreType.DMA,
        ],
    )
    def gather_kernel(tab_hbm, idx_hbm, out_hbm, idx_v, off_v, rows0, rows1,
                      gsem, osem0, osem1):
        wid = lax.axis_index("s") * _NC + lax.axis_index("c")
        base_w = wid * per_w

        pltpu.sync_copy(idx_hbm.at[pl.ds(base_w, per_w)], idx_v)

        lanes = lax.iota(jnp.int32, _L)
        for r in range(PERIOD // _L):
            off_v[pl.ds(r * _L, _L)] = ((lanes + r * _L) % F) * V

        def add_block(j, carry):
            b = j * PERIOD
            for r in range(PERIOD // _L):
                s = pl.ds(b + r * _L, _L)
                idx_v[s] = idx_v[s] + off_v[pl.ds(r * _L, _L)]
            return carry

        lax.fori_loop(0, per_w // PERIOD, add_block, 0)

        rows = [rows0, rows1]
        osems = [osem0, osem1]
        out_cps = [None, None]
        for c in range(n_chunks):
            p = c & 1
            if out_cps[p] is not None:
                out_cps[p].wait()
            cbase = c * CHUNK
            g_cps = [
                pltpu.async_copy(
                    tab_hbm.at[idx_v.at[pl.ds(cbase + r * 128, 128)]],
                    rows[p].at[pl.ds(r * 128, 128)],
                    gsem,
                )
                for r in range(ROWS)
            ]
            for cp in g_cps:
                cp.wait()
            out_cps[p] = pltpu.async_copy(
                rows[p], out_hbm.at[pl.ds(base_w + cbase, CHUNK)], osems[p]
            )
        for cp in out_cps:
            if cp is not None:
                cp.wait()

    return gather_kernel


def kernel(inputs, tables):
    B, F = inputs.shape
    _, V, D = tables.shape
    N = B * F
    tab_t = jnp.transpose(tables, (0, 2, 1))
    tab_lin = _make_transform(F, V, D)(tab_t).reshape(F * V, D)
    idx_flat = inputs.reshape(N)
    out = _make_gather(F, V, D, N)(tab_lin, idx_flat)
    return out.reshape(B, F, D)


# speedup vs baseline: 1.0003x; 1.0003x over previous
import functools

import jax
import jax.numpy as jnp
from jax import lax
from jax.experimental import pallas as pl
from jax.experimental.pallas import tpu as pltpu
from jax.experimental.pallas import tpu_sc as plsc

_info = plsc.get_sparse_core_info()
_NC, _NS, _L = _info.num_cores, _info.num_subcores, _info.num_lanes
_NW = _NC * _NS


def _make_transform(F, V, D):
    TILE = 128
    GCOLS = 512
    JFULL = V // TILE
    VFULL = JFULL * TILE
    VTAIL = V - VFULL
    GPF = -(-VFULL // GCOLS)
    NB = F * GPF
    NRING = 3
    per_w = -(-NB // _NW)
    per_w += (-per_w) % NRING
    n_steps = per_w // NRING
    CHU = 16

    mesh = plsc.VectorSubcoreMesh(core_axis_name="c", subcore_axis_name="s")

    @functools.partial(
        pl.kernel,
        mesh=mesh,
        out_type=jax.ShapeDtypeStruct((F * V * D,), jnp.float32),
        compiler_params=pltpu.CompilerParams(needs_layout_passes=False),
        scratch_types=[
            pltpu.VMEM((D, GCOLS), jnp.float32),
            pltpu.VMEM((D, GCOLS), jnp.float32),
            pltpu.VMEM((D, GCOLS), jnp.float32),
            pltpu.VMEM((GCOLS * D,), jnp.float32),
            pltpu.VMEM((GCOLS * D,), jnp.float32),
            pltpu.VMEM((GCOLS * D,), jnp.float32),
            pltpu.VMEM((D, VTAIL), jnp.float32),
            pltpu.VMEM((VTAIL * D,), jnp.float32),
            pltpu.SemaphoreType.DMA,
            pltpu.SemaphoreType.DMA,
            pltpu.SemaphoreType.DMA,
            pltpu.SemaphoreType.DMA,
            pltpu.SemaphoreType.DMA,
            pltpu.SemaphoreType.DMA,
        ],
    )
    def transform_kernel(tab_t, out_flat, sv0, sv1, sv2, ov0, ov1, ov2,
                         st, ot, is0, is1, is2, os0, os1, os2):
        wid = lax.axis_index("s") * _NC + lax.axis_index("c")
        g0 = wid * per_w
        lanes = lax.iota(jnp.int32, _L)
        zeros = lanes - lanes
        i_lo = lanes
        i_hi = lanes + _L
        isems = [is0, is1, is2]
        osems = [os0, os1, os2]
        svs = [sv0, sv1, sv2]
        ovs = [ov0, ov1, ov2]

        def loc_of(g):
            g = jnp.minimum(g, NB - 1)
            f = g // GPF
            jg = g - f * GPF
            col0 = jnp.minimum(jg * GCOLS, VFULL - GCOLS)
            return f, col0

        def src_of(g):
            f, col0 = loc_of(g)
            return tab_t.at[f, :, pl.ds(col0, GCOLS)]

        def dst_of(g):
            f, col0 = loc_of(g)
            return out_flat.at[pl.ds(f * V * D + col0 * D, GCOLS * D)]

        def transpose_group(s, o):
            def tcol(i, carry):
                ibase = zeros + i * CHU
                for cc in range(CHU):
                    cvec = ibase + cc
                    a = plsc.load_gather(s, [i_lo, cvec])
                    b = plsc.load_gather(s, [i_hi, cvec])
                    o[pl.ds(i * (CHU * D) + cc * D, _L)] = a
                    o[pl.ds(i * (CHU * D) + cc * D + _L, _L)] = b
                return carry

            lax.fori_loop(0, GCOLS // CHU, tcol, 0)

        def transpose_tail(s, o):
            for rr in range(VTAIL):
                cvec = zeros + rr
                a = plsc.load_gather(s, [i_lo, cvec])
                b = plsc.load_gather(s, [i_hi, cvec])
                o[pl.ds(rr * D, _L)] = a
                o[pl.ds(rr * D + _L, _L)] = b

        for p in range(NRING):
            pltpu.async_copy(src_of(g0 + p), svs[p], isems[p])

        def step(u, carry):
            for p in range(NRING):
                g = g0 + NRING * u + p
                pltpu.make_async_copy(src_of(g), svs[p], isems[p]).wait()

                @pl.when(u > 0)
                def _():
                    pltpu.make_async_copy(ovs[p], dst_of(g),
                                          osems[p]).wait()

                transpose_group(svs[p], ovs[p])
                pltpu.async_copy(ovs[p], dst_of(g), osems[p])
                pltpu.async_copy(src_of(g + NRING), svs[p], isems[p])
            return carry

        lax.fori_loop(0, n_steps, step, 0)

        for p in range(NRING):
            pltpu.make_async_copy(src_of(g0), svs[p], isems[p]).wait()
            pltpu.make_async_copy(ovs[p], dst_of(g0), osems[p]).wait()

        wt = jnp.minimum(wid, F - 1)
        pltpu.sync_copy(tab_t.at[wt, :, pl.ds(VFULL, VTAIL)], st)
        transpose_tail(st, ot)
        pltpu.sync_copy(
            ot, out_flat.at[pl.ds(wt * V * D + VFULL * D, VTAIL * D)]
        )

    return transform_kernel


def _make_gather(F, V, D, N):
    assert N % _NW == 0
    per_w = N // _NW
    CHUNK = 1664
    ROWS = CHUNK // 128
    PERIOD = 13 * _L
    assert per_w % CHUNK == 0 and per_w % PERIOD == 0
    n_chunks = per_w // CHUNK

    mesh = plsc.VectorSubcoreMesh(core_axis_name="c", subcore_axis_name="s")

    @functools.partial(
        pl.kernel,
        mesh=mesh,
        out_type=jax.ShapeDtypeStruct((N, D), jnp.float32),
        compiler_params=pltpu.CompilerParams(use_tc_tiling_on_sc=False),
        scratch_types=[
            pltpu.VMEM((per_w,), jnp.int32),
            pltpu.VMEM((PERIOD,), jnp.int32),
            pltpu.VMEM((CHUNK, D), jnp.float32),
            pltpu.VMEM((CHUNK, D), jnp.float32),
            pltpu.SemaphoreType.DMA,
            pltpu.SemaphoreType.DMA,
            pltpu.SemaphoreType.DMA,
        ],
    )
    def gather_kernel(tab_hbm, idx_hbm, out_hbm, idx_v, off_v, rows0, rows1,
                      gsem, osem0, osem1):
        wid = lax.axis_index("s") * _NC + lax.axis_index("c")
        base_w = wid * per_w

        pltpu.sync_copy(idx_hbm.at[pl.ds(base_w, per_w)], idx_v)

        lanes = lax.iota(jnp.int32, _L)
        for r in range(PERIOD // _L):
            off_v[pl.ds(r * _L, _L)] = ((lanes + r * _L) % F) * V

        def add_block(j, carry):
            b = j * PERIOD
            for r in range(PERIOD // _L):
                s = pl.ds(b + r * _L, _L)
                idx_v[s] = idx_v[s] + off_v[pl.ds(r * _L, _L)]
            return carry

        lax.fori_loop(0, per_w // PERIOD, add_block, 0)

        rows = [rows0, rows1]
        osems = [osem0, osem1]
        out_cps = [None, None]
        for c in range(n_chunks):
            p = c & 1
            if out_cps[p] is not None:
                out_cps[p].wait()
            cbase = c * CHUNK
            g_cps = [
                pltpu.async_copy(
                    tab_hbm.at[idx_v.at[pl.ds(cbase + r * 128, 128)]],
                    rows[p].at[pl.ds(r * 128, 128)],
                    gsem,
                )
                for r in range(ROWS)
            ]
            for cp in g_cps:
                cp.wait()
            out_cps[p] = pltpu.async_copy(
                rows[p], out_hbm.at[pl.ds(base_w + cbase, CHUNK)], osems[p]
            )
        for cp in out_cps:
            if cp is not None:
                cp.wait()

    return gather_kernel


def kernel(inputs, tables):
    B, F = inputs.shape
    _, V, D = tables.shape
    N = B * F
    tab_t = jnp.transpose(tables, (0, 2, 1))
    tab_lin = _make_transform(F, V, D)(tab_t).reshape(F * V, D)
    idx_flat = inputs.reshape(N)
    out = _make_gather(F, V, D, N)(tab_lin, idx_flat)
    return out.reshape(B, F, D)


# speedup vs baseline: 2.1057x; 2.1051x over previous
import functools

import jax
import jax.numpy as jnp
from jax import lax
from jax.experimental import pallas as pl
from jax.experimental.pallas import tpu as pltpu
from jax.experimental.pallas import tpu_sc as plsc

_info = plsc.get_sparse_core_info()
_NC, _NS, _L = _info.num_cores, _info.num_subcores, _info.num_lanes
_NW = _NC * _NS


def _make_transform(F, V, D):
    TILE = 128
    GCOLS = 512
    JFULL = V // TILE
    VFULL = JFULL * TILE
    VTAIL = V - VFULL
    GPF = -(-VFULL // GCOLS)
    NB = F * GPF
    NRING = 3
    per_w = -(-NB // _NW)
    per_w += (-per_w) % NRING
    n_steps = per_w // NRING
    CHU = 16

    mesh = plsc.VectorSubcoreMesh(core_axis_name="c", subcore_axis_name="s")

    @functools.partial(
        pl.kernel,
        mesh=mesh,
        out_type=jax.ShapeDtypeStruct((F * V * D,), jnp.float32),
        compiler_params=pltpu.CompilerParams(needs_layout_passes=False),
        scratch_types=[
            pltpu.VMEM((D, GCOLS), jnp.float32),
            pltpu.VMEM((D, GCOLS), jnp.float32),
            pltpu.VMEM((D, GCOLS), jnp.float32),
            pltpu.VMEM((GCOLS * D,), jnp.float32),
            pltpu.VMEM((GCOLS * D,), jnp.float32),
            pltpu.VMEM((GCOLS * D,), jnp.float32),
            pltpu.VMEM((D, VTAIL), jnp.float32),
            pltpu.VMEM((VTAIL * D,), jnp.float32),
            pltpu.SemaphoreType.DMA,
            pltpu.SemaphoreType.DMA,
            pltpu.SemaphoreType.DMA,
            pltpu.SemaphoreType.DMA,
            pltpu.SemaphoreType.DMA,
            pltpu.SemaphoreType.DMA,
        ],
    )
    def transform_kernel(tab_t, out_flat, sv0, sv1, sv2, ov0, ov1, ov2,
                         st, ot, is0, is1, is2, os0, os1, os2):
        wid = lax.axis_index("s") * _NC + lax.axis_index("c")
        g0 = wid * per_w
        lanes = lax.iota(jnp.int32, _L)
        zeros = lanes - lanes
        i_lo = lanes
        i_hi = lanes + _L
        isems = [is0, is1, is2]
        osems = [os0, os1, os2]
        svs = [sv0, sv1, sv2]
        ovs = [ov0, ov1, ov2]

        def loc_of(g):
            g = jnp.minimum(g, NB - 1)
            f = g // GPF
            jg = g - f * GPF
            col0 = jnp.minimum(jg * GCOLS, VFULL - GCOLS)
            return f, col0

        def src_of(g):
            f, col0 = loc_of(g)
            return tab_t.at[f, :, pl.ds(col0, GCOLS)]

        def dst_of(g):
            f, col0 = loc_of(g)
            return out_flat.at[pl.ds(f * V * D + col0 * D, GCOLS * D)]

        diag_c = [(lanes + k) % _L for k in range(_L)]
        diag_w = [((lanes + k) % _L) * D + lanes for k in range(_L)]

        def make_transpose(ncols):
            def transpose(s, o):
                def tcol(i, carry):
                    c0 = i * _L
                    for d0 in range(0, D, _L):
                        rows = lanes + d0
                        sbase = c0 * D + d0
                        for k in range(_L):
                            val = plsc.load_gather(s, [rows, diag_c[k] + c0])
                            plsc.store_scatter(o, [diag_w[k] + sbase], val)
                    return carry

                lax.fori_loop(0, ncols // _L, tcol, 0)

            return transpose

        transpose_group = make_transpose(GCOLS)
        transpose_tail = make_transpose(VTAIL)

        for p in range(NRING):
            pltpu.async_copy(src_of(g0 + p), svs[p], isems[p])

        def step(u, carry):
            for p in range(NRING):
                g = g0 + NRING * u + p
                pltpu.make_async_copy(src_of(g), svs[p], isems[p]).wait()

                @pl.when(u > 0)
                def _():
                    pltpu.make_async_copy(ovs[p], dst_of(g),
                                          osems[p]).wait()

                transpose_group(svs[p], ovs[p])
                pltpu.async_copy(ovs[p], dst_of(g), osems[p])
                pltpu.async_copy(src_of(g + NRING), svs[p], isems[p])
            return carry

        lax.fori_loop(0, n_steps, step, 0)

        for p in range(NRING):
            pltpu.make_async_copy(src_of(g0), svs[p], isems[p]).wait()
            pltpu.make_async_copy(ovs[p], dst_of(g0), osems[p]).wait()

        wt = jnp.minimum(wid, F - 1)
        pltpu.sync_copy(tab_t.at[wt, :, pl.ds(VFULL, VTAIL)], st)
        transpose_tail(st, ot)
        pltpu.sync_copy(
            ot, out_flat.at[pl.ds(wt * V * D + VFULL * D, VTAIL * D)]
        )

    return transform_kernel


def _make_gather(F, V, D, N):
    assert N % _NW == 0
    per_w = N // _NW
    CHUNK = 1664
    ROWS = CHUNK // 128
    PERIOD = 13 * _L
    assert per_w % CHUNK == 0 and per_w % PERIOD == 0
    n_chunks = per_w // CHUNK

    mesh = plsc.VectorSubcoreMesh(core_axis_name="c", subcore_axis_name="s")

    @functools.partial(
        pl.kernel,
        mesh=mesh,
        out_type=jax.ShapeDtypeStruct((N, D), jnp.float32),
        compiler_params=pltpu.CompilerParams(use_tc_tiling_on_sc=False),
        scratch_types=[
            pltpu.VMEM((per_w,), jnp.int32),
            pltpu.VMEM((PERIOD,), jnp.int32),
            pltpu.VMEM((CHUNK, D), jnp.float32),
            pltpu.VMEM((CHUNK, D), jnp.float32),
            pltpu.SemaphoreType.DMA,
            pltpu.SemaphoreType.DMA,
            pltpu.SemaphoreType.DMA,
        ],
    )
    def gather_kernel(tab_hbm, idx_hbm, out_hbm, idx_v, off_v, rows0, rows1,
                      gsem, osem0, osem1):
        wid = lax.axis_index("s") * _NC + lax.axis_index("c")
        base_w = wid * per_w

        pltpu.sync_copy(idx_hbm.at[pl.ds(base_w, per_w)], idx_v)

        lanes = lax.iota(jnp.int32, _L)
        for r in range(PERIOD // _L):
            off_v[pl.ds(r * _L, _L)] = ((lanes + r * _L) % F) * V

        def add_block(j, carry):
            b = j * PERIOD
            for r in range(PERIOD // _L):
                s = pl.ds(b + r * _L, _L)
                idx_v[s] = idx_v[s] + off_v[pl.ds(r * _L, _L)]
            return carry

        lax.fori_loop(0, per_w // PERIOD, add_block, 0)

        rows = [rows0, rows1]
        osems = [osem0, osem1]
        out_cps = [None, None]
        for c in range(n_chunks):
            p = c & 1
            if out_cps[p] is not None:
                out_cps[p].wait()
            cbase = c * CHUNK
            g_cps = [
                pltpu.async_copy(
                    tab_hbm.at[idx_v.at[pl.ds(cbase + r * 128, 128)]],
                    rows[p].at[pl.ds(r * 128, 128)],
                    gsem,
                )
                for r in range(ROWS)
            ]
            for cp in g_cps:
                cp.wait()
            out_cps[p] = pltpu.async_copy(
                rows[p], out_hbm.at[pl.ds(base_w + cbase, CHUNK)], osems[p]
            )
        for cp in out_cps:
            if cp is not None:
                cp.wait()

    return gather_kernel


def kernel(inputs, tables):
    B, F = inputs.shape
    _, V, D = tables.shape
    N = B * F
    tab_t = jnp.transpose(tables, (0, 2, 1))
    tab_lin = _make_transform(F, V, D)(tab_t).reshape(F * V, D)
    idx_flat = inputs.reshape(N)
    out = _make_gather(F, V, D, N)(tab_lin, idx_flat)
    return out.reshape(B, F, D)


# speedup vs baseline: 2.2965x; 1.0906x over previous
import functools

import jax
import jax.numpy as jnp
from jax import lax
from jax.experimental import pallas as pl
from jax.experimental.pallas import tpu as pltpu
from jax.experimental.pallas import tpu_sc as plsc

_info = plsc.get_sparse_core_info()
_NC, _NS, _L = _info.num_cores, _info.num_subcores, _info.num_lanes
_NW = _NC * _NS


def _make_transform(F, V, D):
    TILE = 128
    GCOLS = 896
    JFULL = V // TILE
    VFULL = JFULL * TILE
    VTAIL = V - VFULL
    GPF = -(-VFULL // GCOLS)
    NB = F * GPF
    NRING = 2
    per_w = -(-NB // _NW)
    per_w += (-per_w) % NRING
    n_steps = per_w // NRING
    CHU = 16

    mesh = plsc.VectorSubcoreMesh(core_axis_name="c", subcore_axis_name="s")

    @functools.partial(
        pl.kernel,
        mesh=mesh,
        out_type=jax.ShapeDtypeStruct((F * V * D,), jnp.float32),
        compiler_params=pltpu.CompilerParams(needs_layout_passes=False),
        scratch_types=[
            pltpu.VMEM((D, GCOLS), jnp.float32),
            pltpu.VMEM((D, GCOLS), jnp.float32),
            pltpu.VMEM((GCOLS * D,), jnp.float32),
            pltpu.VMEM((GCOLS * D,), jnp.float32),
            pltpu.VMEM((D, VTAIL), jnp.float32),
            pltpu.VMEM((VTAIL * D,), jnp.float32),
            pltpu.SemaphoreType.DMA,
            pltpu.SemaphoreType.DMA,
            pltpu.SemaphoreType.DMA,
            pltpu.SemaphoreType.DMA,
        ],
    )
    def transform_kernel(tab_t, out_flat, sv0, sv1, ov0, ov1,
                         st, ot, is0, is1, os0, os1):
        wid = lax.axis_index("s") * _NC + lax.axis_index("c")
        g0 = wid * per_w
        lanes = lax.iota(jnp.int32, _L)
        zeros = lanes - lanes
        i_lo = lanes
        i_hi = lanes + _L
        isems = [is0, is1]
        osems = [os0, os1]
        svs = [sv0, sv1]
        ovs = [ov0, ov1]

        def loc_of(g):
            g = jnp.minimum(g, NB - 1)
            f = g // GPF
            jg = g - f * GPF
            col0 = jnp.minimum(jg * GCOLS, VFULL - GCOLS)
            return f, col0

        def src_of(g):
            f, col0 = loc_of(g)
            return tab_t.at[f, :, pl.ds(col0, GCOLS)]

        def dst_of(g):
            f, col0 = loc_of(g)
            return out_flat.at[pl.ds(f * V * D + col0 * D, GCOLS * D)]

        diag_c = [(lanes + k) % _L for k in range(_L)]
        diag_w = [((lanes + k) % _L) * D + lanes for k in range(_L)]

        def make_transpose(ncols):
            def transpose(s, o):
                def tcol(i, carry):
                    c0 = i * _L
                    for d0 in range(0, D, _L):
                        rows = lanes + d0
                        sbase = c0 * D + d0
                        for k in range(_L):
                            val = plsc.load_gather(s, [rows, diag_c[k] + c0])
                            plsc.store_scatter(o, [diag_w[k] + sbase], val)
                    return carry

                lax.fori_loop(0, ncols // _L, tcol, 0)

            return transpose

        transpose_group = make_transpose(GCOLS)
        transpose_tail = make_transpose(VTAIL)

        for p in range(NRING):
            pltpu.async_copy(src_of(g0 + p), svs[p], isems[p])

        def step(u, carry):
            for p in range(NRING):
                g = g0 + NRING * u + p
                pltpu.make_async_copy(src_of(g), svs[p], isems[p]).wait()

                @pl.when(u > 0)
                def _():
                    pltpu.make_async_copy(ovs[p], dst_of(g),
                                          osems[p]).wait()

                transpose_group(svs[p], ovs[p])
                pltpu.async_copy(ovs[p], dst_of(g), osems[p])
                pltpu.async_copy(src_of(g + NRING), svs[p], isems[p])
            return carry

        lax.fori_loop(0, n_steps, step, 0)

        for p in range(NRING):
            pltpu.make_async_copy(src_of(g0), svs[p], isems[p]).wait()
            pltpu.make_async_copy(ovs[p], dst_of(g0), osems[p]).wait()

        wt = jnp.minimum(wid, F - 1)
        pltpu.sync_copy(tab_t.at[wt, :, pl.ds(VFULL, VTAIL)], st)
        transpose_tail(st, ot)
        pltpu.sync_copy(
            ot, out_flat.at[pl.ds(wt * V * D + VFULL * D, VTAIL * D)]
        )

    return transform_kernel


def _make_gather(F, V, D, N, B):
    assert N % _NW == 0
    per_w = N // _NW
    CHUNK = 512
    ROWS = CHUNK // 128
    assert per_w % CHUNK == 0 and B % CHUNK == 0
    n_chunks = per_w // CHUNK

    mesh = plsc.VectorSubcoreMesh(core_axis_name="c", subcore_axis_name="s")

    @functools.partial(
        pl.kernel,
        mesh=mesh,
        out_type=jax.ShapeDtypeStruct((F * D, B), jnp.float32),
        compiler_params=pltpu.CompilerParams(
            use_tc_tiling_on_sc=False, needs_layout_passes=False
        ),
        scratch_types=[
            pltpu.VMEM((per_w,), jnp.int32),
            pltpu.VMEM((CHUNK, D), jnp.float32),
            pltpu.VMEM((CHUNK, D), jnp.float32),
            pltpu.VMEM((D, CHUNK), jnp.float32),
            pltpu.VMEM((D, CHUNK), jnp.float32),
            pltpu.SemaphoreType.DMA,
            pltpu.SemaphoreType.DMA,
            pltpu.SemaphoreType.DMA,
        ],
    )
    def gather_kernel(tab_hbm, idx_hbm, out_hbm, idx_v, rows0, rows1,
                      tr0, tr1, gsem, osem0, osem1):
        wid = lax.axis_index("s") * _NC + lax.axis_index("c")
        base_w = wid * per_w

        pltpu.sync_copy(idx_hbm.at[pl.ds(base_w, per_w)], idx_v)

        lanes = lax.iota(jnp.int32, _L)

        shift = B.bit_length() - 1
        assert B == 1 << shift

        def add_block(j, carry):
            b = j * _L
            pvec = (base_w + b) + lanes
            idx_v[pl.ds(b, _L)] = (
                idx_v[pl.ds(b, _L)]
                + lax.shift_right_logical(pvec, shift) * V
            )
            return carry

        lax.fori_loop(0, per_w // _L, add_block, 0)

        diag_c = [(lanes + k) % _L for k in range(_L)]

        def transpose_rows(s, o):
            def tcol(i, carry):
                b0 = i * _L
                brow = lanes + b0
                for d0 in range(0, D, _L):
                    for k in range(_L):
                        dcol = diag_c[k] + d0
                        val = plsc.load_gather(s, [brow, dcol])
                        plsc.store_scatter(o, [dcol, brow], val)
                return carry

            lax.fori_loop(0, CHUNK // _L, tcol, 0)

        rows = [rows0, rows1]
        trs = [tr0, tr1]
        osems = [osem0, osem1]

        def do_chunk(c, p, u):
            cbase = c * CHUNK
            pos0 = base_w + cbase
            f = pos0 // B
            bb = pos0 - f * B
            dst = out_hbm.at[pl.ds(f * D, D), pl.ds(bb, CHUNK)]
            g_cps = [
                pltpu.async_copy(
                    tab_hbm.at[idx_v.at[pl.ds(cbase + r * 128, 128)]],
                    rows[p].at[pl.ds(r * 128, 128)],
                    gsem,
                )
                for r in range(ROWS)
            ]

            @pl.when(u > 0)
            def _():
                pltpu.make_async_copy(trs[p], dst, osems[p]).wait()

            for cp in g_cps:
                cp.wait()
            transpose_rows(rows[p], trs[p])
            pltpu.async_copy(trs[p], dst, osems[p])

        def step(u, carry):
            do_chunk(2 * u, 0, u)
            do_chunk(2 * u + 1, 1, u)
            return carry

        lax.fori_loop(0, n_chunks // 2, step, 0)

        for p in range(2):
            pltpu.make_async_copy(
                trs[p], out_hbm.at[pl.ds(0, D), pl.ds(0, CHUNK)], osems[p]
            ).wait()

    return gather_kernel


def kernel(inputs, tables):
    B, F = inputs.shape
    _, V, D = tables.shape
    N = B * F
    tab_t = jnp.transpose(tables, (0, 2, 1))
    tab_lin = _make_transform(F, V, D)(tab_t).reshape(F * V, D)
    idx_flat = inputs.T.reshape(N)
    out = _make_gather(F, V, D, N, B)(tab_lin, idx_flat)
    return out.reshape(F, D, B).transpose(2, 0, 1)


# speedup vs baseline: 4.9228x; 2.1436x over previous
import functools

import jax
import jax.numpy as jnp
from jax import lax
from jax.experimental import pallas as pl
from jax.experimental.pallas import tpu as pltpu
from jax.experimental.pallas import tpu_sc as plsc

_info = plsc.get_sparse_core_info()
_NC, _NS, _L = _info.num_cores, _info.num_subcores, _info.num_lanes
_NW = _NC * _NS


def _make_transform(F, V, D):
    TILE = 128
    GCOLS = 896
    JFULL = V // TILE
    VFULL = JFULL * TILE
    VTAIL = V - VFULL
    GPF = -(-VFULL // GCOLS)
    NB = F * GPF
    NRING = 2
    per_w = -(-NB // _NW)
    per_w += (-per_w) % NRING
    n_steps = per_w // NRING
    CHU = 16

    mesh = plsc.VectorSubcoreMesh(core_axis_name="c", subcore_axis_name="s")

    @functools.partial(
        pl.kernel,
        mesh=mesh,
        out_type=jax.ShapeDtypeStruct((F * V * D,), jnp.float32),
        compiler_params=pltpu.CompilerParams(needs_layout_passes=False),
        scratch_types=[
            pltpu.VMEM((D, GCOLS), jnp.float32),
            pltpu.VMEM((D, GCOLS), jnp.float32),
            pltpu.VMEM((GCOLS * D,), jnp.float32),
            pltpu.VMEM((GCOLS * D,), jnp.float32),
            pltpu.VMEM((D, VTAIL), jnp.float32),
            pltpu.VMEM((VTAIL * D,), jnp.float32),
            pltpu.SemaphoreType.DMA,
            pltpu.SemaphoreType.DMA,
            pltpu.SemaphoreType.DMA,
            pltpu.SemaphoreType.DMA,
        ],
    )
    def transform_kernel(tab_t, out_flat, sv0, sv1, ov0, ov1,
                         st, ot, is0, is1, os0, os1):
        wid = lax.axis_index("s") * _NC + lax.axis_index("c")
        g0 = wid * per_w
        lanes = lax.iota(jnp.int32, _L)
        zeros = lanes - lanes
        i_lo = lanes
        i_hi = lanes + _L
        isems = [is0, is1]
        osems = [os0, os1]
        svs = [sv0, sv1]
        ovs = [ov0, ov1]

        def loc_of(g):
            g = jnp.minimum(g, NB - 1)
            f = g // GPF
            jg = g - f * GPF
            col0 = jnp.minimum(jg * GCOLS, VFULL - GCOLS)
            return f, col0

        def src_of(g):
            f, col0 = loc_of(g)
            return tab_t.at[f, :, pl.ds(col0, GCOLS)]

        def dst_of(g):
            f, col0 = loc_of(g)
            return out_flat.at[pl.ds(f * V * D + col0 * D, GCOLS * D)]

        diag_c = [(lanes + k) % _L for k in range(_L)]
        diag_w = [((lanes + k) % _L) * D + lanes for k in range(_L)]

        BT = 4

        def make_transpose(ncols):
            def transpose(s, o):
                def tcol(i, carry):
                    c0 = i * _L
                    for d0 in range(0, D, _L):
                        rows = lanes + d0
                        sbase = c0 * D + d0
                        for kb in range(0, _L, BT):
                            vals = [
                                plsc.load_gather(s, [rows, diag_c[kb + t] + c0])
                                for t in range(BT)
                            ]
                            for t in range(BT):
                                plsc.store_scatter(
                                    o, [diag_w[kb + t] + sbase], vals[t]
                                )
                    return carry

                lax.fori_loop(0, ncols // _L, tcol, 0)

            return transpose

        transpose_group = make_transpose(GCOLS)
        transpose_tail = make_transpose(VTAIL)

        for p in range(NRING):
            pltpu.async_copy(src_of(g0 + p), svs[p], isems[p])

        def step(u, carry):
            for p in range(NRING):
                g = g0 + NRING * u + p
                pltpu.make_async_copy(src_of(g), svs[p], isems[p]).wait()

                @pl.when(u > 0)
                def _():
                    pltpu.make_async_copy(ovs[p], dst_of(g),
                                          osems[p]).wait()

                transpose_group(svs[p], ovs[p])
                pltpu.async_copy(ovs[p], dst_of(g), osems[p])
                pltpu.async_copy(src_of(g + NRING), svs[p], isems[p])
            return carry

        lax.fori_loop(0, n_steps, step, 0)

        for p in range(NRING):
            pltpu.make_async_copy(src_of(g0), svs[p], isems[p]).wait()
            pltpu.make_async_copy(ovs[p], dst_of(g0), osems[p]).wait()

        wt = jnp.minimum(wid, F - 1)
        pltpu.sync_copy(tab_t.at[wt, :, pl.ds(VFULL, VTAIL)], st)
        transpose_tail(st, ot)
        pltpu.sync_copy(
            ot, out_flat.at[pl.ds(wt * V * D + VFULL * D, VTAIL * D)]
        )

    return transform_kernel


def _make_gather(F, V, D, N, B):
    assert N % _NW == 0
    per_w = N // _NW
    CHUNK = 512
    ROWS = CHUNK // 128
    assert per_w % CHUNK == 0 and B % CHUNK == 0
    n_chunks = per_w // CHUNK

    mesh = plsc.VectorSubcoreMesh(core_axis_name="c", subcore_axis_name="s")

    @functools.partial(
        pl.kernel,
        mesh=mesh,
        out_type=jax.ShapeDtypeStruct((F * D, B), jnp.float32),
        compiler_params=pltpu.CompilerParams(
            use_tc_tiling_on_sc=False, needs_layout_passes=False
        ),
        scratch_types=[
            pltpu.VMEM((per_w,), jnp.int32),
            pltpu.VMEM((CHUNK, D), jnp.float32),
            pltpu.VMEM((CHUNK, D), jnp.float32),
            pltpu.VMEM((D, CHUNK), jnp.float32),
            pltpu.VMEM((D, CHUNK), jnp.float32),
            pltpu.SemaphoreType.DMA,
            pltpu.SemaphoreType.DMA,
            pltpu.SemaphoreType.DMA,
        ],
    )
    def gather_kernel(tab_hbm, idx_hbm, out_hbm, idx_v, rows0, rows1,
                      tr0, tr1, gsem, osem0, osem1):
        wid = lax.axis_index("s") * _NC + lax.axis_index("c")
        base_w = wid * per_w

        pltpu.sync_copy(idx_hbm.at[pl.ds(base_w, per_w)], idx_v)

        lanes = lax.iota(jnp.int32, _L)

        shift = B.bit_length() - 1
        assert B == 1 << shift

        def add_block(j, carry):
            b = j * _L
            pvec = (base_w + b) + lanes
            idx_v[pl.ds(b, _L)] = (
                idx_v[pl.ds(b, _L)]
                + lax.shift_right_logical(pvec, shift) * V
            )
            return carry

        lax.fori_loop(0, per_w // _L, add_block, 0)

        diag_c = [(lanes + k) % _L for k in range(_L)]

        BT = 4

        def transpose_rows(s, o):
            def tcol(i, carry):
                b0 = i * _L
                brow = lanes + b0
                for d0 in range(0, D, _L):
                    for kb in range(0, _L, BT):
                        dcols = [diag_c[kb + t] + d0 for t in range(BT)]
                        vals = [
                            plsc.load_gather(s, [brow, dcols[t]])
                            for t in range(BT)
                        ]
                        for t in range(BT):
                            plsc.store_scatter(o, [dcols[t], brow], vals[t])
                return carry

            lax.fori_loop(0, CHUNK // _L, tcol, 0)

        rows = [rows0, rows1]
        trs = [tr0, tr1]
        osems = [osem0, osem1]

        def do_chunk(c, p, u):
            cbase = c * CHUNK
            pos0 = base_w + cbase
            f = pos0 // B
            bb = pos0 - f * B
            dst = out_hbm.at[pl.ds(f * D, D), pl.ds(bb, CHUNK)]
            g_cps = [
                pltpu.async_copy(
                    tab_hbm.at[idx_v.at[pl.ds(cbase + r * 128, 128)]],
                    rows[p].at[pl.ds(r * 128, 128)],
                    gsem,
                )
                for r in range(ROWS)
            ]

            @pl.when(u > 0)
            def _():
                pltpu.make_async_copy(trs[p], dst, osems[p]).wait()

            for cp in g_cps:
                cp.wait()
            transpose_rows(rows[p], trs[p])
            pltpu.async_copy(trs[p], dst, osems[p])

        def step(u, carry):
            do_chunk(2 * u, 0, u)
            do_chunk(2 * u + 1, 1, u)
            return carry

        lax.fori_loop(0, n_chunks // 2, step, 0)

        for p in range(2):
            pltpu.make_async_copy(
                trs[p], out_hbm.at[pl.ds(0, D), pl.ds(0, CHUNK)], osems[p]
            ).wait()

    return gather_kernel


def kernel(inputs, tables):
    B, F = inputs.shape
    _, V, D = tables.shape
    N = B * F
    tab_t = jnp.transpose(tables, (0, 2, 1))
    tab_lin = _make_transform(F, V, D)(tab_t).reshape(F * V, D)
    idx_flat = inputs.T.reshape(N)
    out = _make_gather(F, V, D, N, B)(tab_lin, idx_flat)
    return out.reshape(F, D, B).transpose(2, 0, 1)
